# baseline (speedup 1.0000x reference)
import functools

import jax
import jax.numpy as jnp
from jax import lax
from jax.experimental import pallas as pl
from jax.experimental.pallas import tpu as pltpu
from jax.experimental.pallas import tpu_sc as plsc

_info = plsc.get_sparse_core_info()
_NC = _info.num_cores
_NS = _info.num_subcores
_NW = _NC * _NS


@jax.jit
def _sc_embed(table, idx_cm):
    D = table.shape[1]
    assert D == 64 and idx_cm.shape[0] == 409600
    n_units = 3200
    upw = n_units // _NW

    mesh = plsc.VectorSubcoreMesh(core_axis_name="c", subcore_axis_name="s")

    @functools.partial(
        pl.kernel,
        out_type=jax.ShapeDtypeStruct((20, 20, 8, 8, 8, 128), jnp.float32),
        mesh=mesh,
        scratch_types=(
            [pltpu.VMEM((128,), jnp.int32) for _ in range(2)]
            + [pltpu.VMEM((128, 64), jnp.float32) for _ in range(2)]
            + [pltpu.VMEM((8, 8, 128), jnp.float32) for _ in range(2)]
            + [pltpu.SemaphoreType.DMA for _ in range(4)]
        ),
        compiler_params=pltpu.CompilerParams(needs_layout_passes=False),
    )
    def body(table_hbm, idx_hbm, out6, *bufs):
        idxv = bufs[0:2]
        rows = bufs[2:4]
        tbuf = bufs[4:6]
        gsem = bufs[6:8]
        wsem = bufs[8:10]
        wid = lax.axis_index("s") * _NC + lax.axis_index("c")
        base_u = wid * upw
        lanes = lax.iota(jnp.int32, 16)

        def unit(k, p):
            u = base_u + k
            cell = u // 8
            bb = u % 8
            h = cell // 20
            w = cell % 20
            pltpu.sync_copy(idx_hbm.at[pl.ds(cell * 1024 + bb * 128, 128)],
                            idxv[p])
            def grp(g, carry):
                vec = idxv[p][pl.ds(g * 16, 16)]
                for j in range(16):
                    v = vec[j]
                    pltpu.async_copy(
                        table_hbm.at[pl.ds(v, 1), :],
                        rows[p].at[pl.ds(g * 16 + j, 1), :], gsem[p])
                return carry
            lax.fori_loop(0, 8, grp, 0)
            pltpu.make_async_copy(
                table_hbm.at[pl.ds(0, 128), :], rows[p], gsem[p]).wait()
            @pl.when(k >= 2)
            def _():
                pltpu.make_async_copy(
                    tbuf[p], out6.at[0, 0, :, 0], wsem[p]).wait()
            def trn(cb, carry):
                for ci in range(8):
                    c = jnp.full((16,), ci, jnp.int32) + cb * 8
                    for g in range(8):
                        bi = lanes + g * 16
                        vec = plsc.load_gather(rows[p], [bi, c])
                        tbuf[p][cb, ci, pl.ds(g * 16, 16)] = vec
                return carry
            lax.fori_loop(0, 8, trn, 0)
            pltpu.async_copy(tbuf[p], out6.at[h, w, :, bb], wsem[p])

        def pair(k2, carry):
            unit(k2 * 2, 0)
            unit(k2 * 2 + 1, 1)
            return carry

        lax.fori_loop(0, upw // 2, pair, 0)
        for p in range(2):
            pltpu.make_async_copy(
                tbuf[p], out6.at[0, 0, :, 0], wsem[p]).wait()

    return body(table, idx_cm)


def kernel(x, table):
    assert x.ndim == 4, f"TokenEmbedding expects 4D [B,H,W,C], got {x.shape}"
    vocab, dim = table.shape
    if x.shape[-1] == vocab:
        idx = jnp.argmax(x, axis=-1).astype(jnp.int32)
    else:
        idx = x.astype(jnp.int32)
    B, H, W = x.shape[0], x.shape[1], x.shape[2]
    idx_cm = idx.reshape(B, H * W).T.reshape(-1)
    out6 = _sc_embed(table, idx_cm)
    out = out6.transpose(3, 5, 0, 1, 2, 4).reshape(B, H, W, dim)
    return out


# speedup vs baseline: 1.2530x; 1.2530x over previous
import functools

import jax
import jax.numpy as jnp
from jax import lax
from jax.experimental import pallas as pl
from jax.experimental.pallas import tpu as pltpu
from jax.experimental.pallas import tpu_sc as plsc

_info = plsc.get_sparse_core_info()
_NC = _info.num_cores
_NS = _info.num_subcores
_NW = _NC * _NS


@functools.partial(jax.jit, static_argnames=("cell", "nbuf"))
def _sc_gather(table, idx, *, cell=400, nbuf=2):
    B = idx.shape[0]
    D = table.shape[1]
    NB = B // cell
    b_per_w = NB // _NW
    n_groups = b_per_w // nbuf
    assert NB * cell == B and n_groups * nbuf == b_per_w

    mesh = plsc.VectorSubcoreMesh(core_axis_name="c", subcore_axis_name="s")

    @functools.partial(
        pl.kernel,
        out_type=jax.ShapeDtypeStruct((NB, cell, D), jnp.float32),
        mesh=mesh,
        scratch_types=(
            [pltpu.VMEM((b_per_w * cell,), jnp.int32)]
            + [pltpu.VMEM((cell, D), jnp.float32) for _ in range(nbuf)]
            + [pltpu.SemaphoreType.DMA for _ in range(2 * nbuf)]
        ),
        compiler_params=pltpu.CompilerParams(use_tc_tiling_on_sc=False),
    )
    def body(table_hbm, idx_hbm, out_hbm, idx_all, *bufs):
        rows = bufs[:nbuf]
        gsem = bufs[nbuf:2 * nbuf]
        wsem = bufs[2 * nbuf:]
        wid = lax.axis_index("s") * _NC + lax.axis_index("c")
        base = wid * b_per_w

        pltpu.sync_copy(idx_hbm.at[pl.ds(base * cell, b_per_w * cell)],
                        idx_all)

        def start_gather(i, b):
            src = table_hbm.at[idx_all.at[pl.ds(i * cell, cell)]]
            return pltpu.async_copy(src, rows[b], gsem[b])

        for b in range(nbuf):
            start_gather(b, b)

        def group(g, carry):
            for b in range(nbuf):
                i = g * nbuf + b
                pltpu.make_async_copy(
                    table_hbm.at[idx_all.at[pl.ds(i * cell, cell)]],
                    rows[b], gsem[b]).wait()
                pltpu.async_copy(rows[b], out_hbm.at[base + i], wsem[b])

                @pl.when(g < n_groups - 1)
                def _():
                    pltpu.make_async_copy(
                        rows[b], out_hbm.at[base + i], wsem[b]).wait()
                    start_gather(i + nbuf, b)
            return carry

        lax.fori_loop(0, n_groups, group, 0)

        for b in range(nbuf):
            i = (n_groups - 1) * nbuf + b
            pltpu.make_async_copy(
                rows[b], out_hbm.at[base + i], wsem[b]).wait()

    return body(table, idx)


def kernel(x, table):
    assert x.ndim == 4, f"TokenEmbedding expects 4D [B,H,W,C], got {x.shape}"
    vocab, dim = table.shape
    if x.shape[-1] == vocab:
        idx = jnp.argmax(x, axis=-1).astype(jnp.int32)
    else:
        idx = x.astype(jnp.int32)
    B, H, W = x.shape[0], x.shape[1], x.shape[2]
    flat = idx.reshape(-1)
    out3 = _sc_gather(table, flat, cell=H * W)
    return out3.reshape(B, H, W, dim)


# speedup vs baseline: 1.6197x; 1.2927x over previous
import functools

import jax
import jax.numpy as jnp
from jax import lax
from jax.experimental import pallas as pl
from jax.experimental.pallas import tpu as pltpu
from jax.experimental.pallas import tpu_sc as plsc

_info = plsc.get_sparse_core_info()
_NC = _info.num_cores
_NS = _info.num_subcores
_NW = _NC * _NS


@jax.jit
def _sc_embed(table, idx_cm):
    D = table.shape[1]
    assert D == 64 and idx_cm.shape[0] == 409600
    n_units = 3200
    upw = n_units // _NW

    mesh = plsc.VectorSubcoreMesh(core_axis_name="c", subcore_axis_name="s")

    @functools.partial(
        pl.kernel,
        out_type=jax.ShapeDtypeStruct((1024, 400, 64), jnp.float32),
        mesh=mesh,
        scratch_types=(
            [pltpu.VMEM((128,), jnp.int32) for _ in range(2)]
            + [pltpu.VMEM((128, 64), jnp.float32) for _ in range(2)]
            + [pltpu.SemaphoreType.DMA for _ in range(4)]
        ),
        compiler_params=pltpu.CompilerParams(needs_layout_passes=False),
    )
    def body(table_hbm, idx_hbm, out3, *bufs):
        idxv = bufs[0:2]
        rows = bufs[2:4]
        gsem = bufs[4:6]
        wsem = bufs[6:8]
        wid = lax.axis_index("s") * _NC + lax.axis_index("c")
        base_u = wid * upw
        lanes = lax.iota(jnp.int32, 16)

        def unit(k, p):
            u = base_u + k
            cell = u // 8
            bb = u % 8
            h = cell // 20
            w = cell % 20
            pltpu.sync_copy(idx_hbm.at[pl.ds(cell * 1024 + bb * 128, 128)],
                            idxv[p])
            @pl.when(k >= 2)
            def _():
                pltpu.make_async_copy(
                    rows[p], out3.at[pl.ds(0, 128), 0], wsem[p]).wait()
            def grp(g, carry):
                vec = idxv[p][pl.ds(g * 16, 16)]
                for j in range(16):
                    v = vec[j]
                    pltpu.async_copy(
                        table_hbm.at[pl.ds(v, 1), :],
                        rows[p].at[pl.ds(g * 16 + j, 1), :], gsem[p])
                return carry
            lax.fori_loop(0, 8, grp, 0)
            pltpu.make_async_copy(
                table_hbm.at[pl.ds(0, 128), :], rows[p], gsem[p]).wait()
            pltpu.async_copy(rows[p], out3.at[pl.ds(bb * 128, 128), cell],
                             wsem[p])

        def pair(k2, carry):
            unit(k2 * 2, 0)
            unit(k2 * 2 + 1, 1)
            return carry

        lax.fori_loop(0, upw // 2, pair, 0)
        for p in range(2):
            pltpu.make_async_copy(
                rows[p], out3.at[pl.ds(0, 128), 0], wsem[p]).wait()

    return body(table, idx_cm)


def kernel(x, table):
    assert x.ndim == 4, f"TokenEmbedding expects 4D [B,H,W,C], got {x.shape}"
    vocab, dim = table.shape
    if x.shape[-1] == vocab:
        idx = jnp.argmax(x, axis=-1).astype(jnp.int32)
    else:
        idx = x.astype(jnp.int32)
    B, H, W = x.shape[0], x.shape[1], x.shape[2]
    idx_cm = idx.reshape(B, H * W).T.reshape(-1)
    out3 = _sc_embed(table, idx_cm)
    out = out3.reshape(B, H, W, dim)
    return out


# speedup vs baseline: 1.7565x; 1.0845x over previous
import functools

import jax
import jax.numpy as jnp
from jax import lax
from jax.experimental import pallas as pl
from jax.experimental.pallas import tpu as pltpu
from jax.experimental.pallas import tpu_sc as plsc

_info = plsc.get_sparse_core_info()
_NC = _info.num_cores
_NS = _info.num_subcores
_NW = _NC * _NS


@jax.jit
def _sc_embed(table, idx_cm):
    D = table.shape[1]
    assert D == 64 and idx_cm.shape[0] == 409600
    n_units = 1600
    upw = n_units // _NW

    mesh = plsc.VectorSubcoreMesh(core_axis_name="c", subcore_axis_name="s")

    @functools.partial(
        pl.kernel,
        out_type=jax.ShapeDtypeStruct((1024, 400, 64), jnp.float32),
        mesh=mesh,
        scratch_types=(
            [pltpu.VMEM((256,), jnp.int32) for _ in range(2)]
            + [pltpu.VMEM((256, 64), jnp.float32) for _ in range(2)]
            + [pltpu.SemaphoreType.DMA for _ in range(4)]
        ),
        compiler_params=pltpu.CompilerParams(needs_layout_passes=False),
    )
    def body(table_hbm, idx_hbm, out3, *bufs):
        idxv = bufs[0:2]
        rows = bufs[2:4]
        gsem = bufs[4:6]
        wsem = bufs[6:8]
        wid = lax.axis_index("s") * _NC + lax.axis_index("c")
        base_u = wid * upw
        lanes = lax.iota(jnp.int32, 16)

        def unit(k, p):
            u = base_u + k
            cell = u // 4
            bb = u % 4
            pltpu.sync_copy(idx_hbm.at[pl.ds(cell * 1024 + bb * 256, 256)],
                            idxv[p])
            @pl.when(k >= 2)
            def _():
                pltpu.make_async_copy(
                    rows[p], out3.at[pl.ds(0, 256), 0], wsem[p]).wait()
            def grp(g, carry):
                vec = idxv[p][pl.ds(g * 16, 16)]
                for j in range(16):
                    v = vec[j]
                    pltpu.async_copy(
                        table_hbm.at[pl.ds(v, 1), :],
                        rows[p].at[pl.ds(g * 16 + j, 1), :], gsem[p])
                return carry
            lax.fori_loop(0, 16, grp, 0)
            pltpu.make_async_copy(
                table_hbm.at[pl.ds(0, 256), :], rows[p], gsem[p]).wait()
            pltpu.async_copy(rows[p], out3.at[pl.ds(bb * 256, 256), cell],
                             wsem[p])

        def pair(k2, carry):
            unit(k2 * 2, 0)
            unit(k2 * 2 + 1, 1)
            return carry

        lax.fori_loop(0, upw // 2, pair, 0)
        for p in range(2):
            pltpu.make_async_copy(
                rows[p], out3.at[pl.ds(0, 256), 0], wsem[p]).wait()

    return body(table, idx_cm)


def kernel(x, table):
    assert x.ndim == 4, f"TokenEmbedding expects 4D [B,H,W,C], got {x.shape}"
    vocab, dim = table.shape
    if x.shape[-1] == vocab:
        idx = jnp.argmax(x, axis=-1).astype(jnp.int32)
    else:
        idx = x.astype(jnp.int32)
    B, H, W = x.shape[0], x.shape[1], x.shape[2]
    idx_cm = idx.reshape(B, H * W).T.reshape(-1)
    out3 = _sc_embed(table, idx_cm)
    out = out3.reshape(B, H, W, dim)
    return out


# speedup vs baseline: 1.7998x; 1.0247x over previous
import functools

import jax
import jax.numpy as jnp
from jax import lax
from jax.experimental import pallas as pl
from jax.experimental.pallas import tpu as pltpu
from jax.experimental.pallas import tpu_sc as plsc

_info = plsc.get_sparse_core_info()
_NC = _info.num_cores
_NS = _info.num_subcores
_NW = _NC * _NS


@jax.jit
def _sc_embed(table, idx_cm):
    D = table.shape[1]
    assert D == 64 and idx_cm.shape[0] == 409600
    n_units = 1600
    upw = n_units // _NW

    mesh = plsc.VectorSubcoreMesh(core_axis_name="c", subcore_axis_name="s")

    @functools.partial(
        pl.kernel,
        out_type=jax.ShapeDtypeStruct((1024, 400, 64), jnp.float32),
        mesh=mesh,
        scratch_types=(
            [pltpu.VMEM((256,), jnp.int32) for _ in range(2)]
            + [pltpu.VMEM((256, 64), jnp.float32) for _ in range(2)]
            + [pltpu.SemaphoreType.DMA for _ in range(4)]
        ),
        compiler_params=pltpu.CompilerParams(needs_layout_passes=False),
    )
    def body(table_hbm, idx_hbm, out3, *bufs):
        idxv = bufs[0:2]
        rows = bufs[2:4]
        gsem = bufs[4:6]
        wsem = bufs[6:8]
        wid = lax.axis_index("s") * _NC + lax.axis_index("c")
        base_u = wid * upw
        lanes = lax.iota(jnp.int32, 16)

        def stage(k, p):
            u = base_u + k
            cell = u // 4
            bb = u % 4
            pltpu.sync_copy(idx_hbm.at[pl.ds(cell * 1024 + bb * 256, 256)],
                            idxv[p])
            @pl.when(k >= 2)
            def _():
                pltpu.make_async_copy(
                    rows[p], out3.at[pl.ds(0, 256), 0], wsem[p]).wait()

            def grp(g, carry):
                vec = idxv[p][pl.ds(g * 16, 16)]
                for j in range(16):
                    v = vec[j]
                    pltpu.async_copy(
                        table_hbm.at[pl.ds(v, 1), :],
                        rows[p].at[pl.ds(g * 16 + j, 1), :], gsem[p])
                return carry
            lax.fori_loop(0, 16, grp, 0)

        def finish(k, p):
            u = base_u + k
            cell = u // 4
            bb = u % 4
            pltpu.make_async_copy(
                table_hbm.at[pl.ds(0, 256), :], rows[p], gsem[p]).wait()
            pltpu.async_copy(rows[p], out3.at[pl.ds(bb * 256, 256), cell],
                             wsem[p])

        stage(0, 0)

        def pair(k2, carry):
            k = k2 * 2
            stage(k + 1, 1)
            finish(k, 0)

            @pl.when(k2 < upw // 2 - 1)
            def _():
                stage(k + 2, 0)
            finish(k + 1, 1)
            return carry

        lax.fori_loop(0, upw // 2, pair, 0)
        for p in range(2):
            pltpu.make_async_copy(
                rows[p], out3.at[pl.ds(0, 256), 0], wsem[p]).wait()

    return body(table, idx_cm)


def kernel(x, table):
    assert x.ndim == 4, f"TokenEmbedding expects 4D [B,H,W,C], got {x.shape}"
    vocab, dim = table.shape
    if x.shape[-1] == vocab:
        idx = jnp.argmax(x, axis=-1).astype(jnp.int32)
    else:
        idx = x.astype(jnp.int32)
    B, H, W = x.shape[0], x.shape[1], x.shape[2]
    idx_cm = idx.reshape(B, H * W).T.reshape(-1)
    out3 = _sc_embed(table, idx_cm)
    out = out3.reshape(B, H, W, dim)
    return out
